# baseline (speedup 1.0000x reference)
import jax
import jax.numpy as jnp
from jax.experimental import pallas as pl
from jax.experimental.pallas import tpu as pltpu

Q_LEN = 2048
K_LEN = 2048
EMB = 64
BI = 8


def _body(trev_ref, out_ref):
    i0 = pl.program_id(0) * BI
    for r in range(BI):
        w = K_LEN - 1 - (i0 + r)
        rolled = pltpu.roll(trev_ref[...], -w, axis=1)
        out_ref[r] = rolled[:, :K_LEN]


def kernel(q, k, rel_emb):
    trev = jnp.flip(rel_emb, axis=0).T
    out_t = pl.pallas_call(
        _body,
        grid=(Q_LEN // BI,),
        in_specs=[
            pl.BlockSpec((EMB, 2 * K_LEN), lambda g: (0, 0),
                         memory_space=pltpu.VMEM),
        ],
        out_specs=pl.BlockSpec((BI, EMB, K_LEN), lambda g: (g, 0, 0)),
        out_shape=jax.ShapeDtypeStruct((Q_LEN, EMB, K_LEN), jnp.float32),
    )(trev)
    return jnp.transpose(out_t, (0, 2, 1))


# speedup vs baseline: 1.9348x; 1.9348x over previous
import jax
import jax.numpy as jnp
from jax.experimental import pallas as pl
from jax.experimental.pallas import tpu as pltpu

Q_LEN = 2048
K_LEN = 2048
EMB = 64
BI = 16


def _body(trev_ref, out_ref):
    i0 = pl.program_id(0) * BI
    w_min = K_LEN - 1 - (i0 + BI - 1)
    rolled = pltpu.roll(trev_ref[...], -w_min, axis=1)
    for r in range(BI):
        d = BI - 1 - r
        out_ref[r] = rolled[:, d:d + K_LEN]


def kernel(q, k, rel_emb):
    trev = jnp.flip(rel_emb, axis=0).T
    out_t = pl.pallas_call(
        _body,
        grid=(Q_LEN // BI,),
        in_specs=[
            pl.BlockSpec((EMB, 2 * K_LEN), lambda g: (0, 0),
                         memory_space=pltpu.VMEM),
        ],
        out_specs=pl.BlockSpec((BI, EMB, K_LEN), lambda g: (g, 0, 0)),
        out_shape=jax.ShapeDtypeStruct((Q_LEN, EMB, K_LEN), jnp.float32),
    )(trev)
    return jnp.transpose(out_t, (0, 2, 1))


# speedup vs baseline: 2.0851x; 1.0776x over previous
import jax
import jax.numpy as jnp
from jax.experimental import pallas as pl
from jax.experimental.pallas import tpu as pltpu

Q_LEN = 2048
K_LEN = 2048
EMB = 64
BI = 32


def _body(trev_ref, out_ref):
    i0 = pl.program_id(0) * BI
    w_min = K_LEN - 1 - (i0 + BI - 1)
    rolled = pltpu.roll(trev_ref[...], -w_min, axis=1)
    for r in range(BI):
        d = BI - 1 - r
        out_ref[r] = rolled[:, d:d + K_LEN]


def kernel(q, k, rel_emb):
    trev = jnp.flip(rel_emb, axis=0).T
    out_t = pl.pallas_call(
        _body,
        grid=(Q_LEN // BI,),
        in_specs=[
            pl.BlockSpec((EMB, 2 * K_LEN), lambda g: (0, 0),
                         memory_space=pltpu.VMEM),
        ],
        out_specs=pl.BlockSpec((BI, EMB, K_LEN), lambda g: (g, 0, 0)),
        out_shape=jax.ShapeDtypeStruct((Q_LEN, EMB, K_LEN), jnp.float32),
    )(trev)
    return jnp.transpose(out_t, (0, 2, 1))
